# Initial kernel scaffold; baseline (speedup 1.0000x reference)
import jax
import jax.numpy as jnp
from jax.experimental import pallas as pl


def kernel(node_embeddings, edge_index, edge_types, emb0, Ws0, bs0, Wm0, bm0, g0, be0, emb1, Ws1, bs1, Wm1, bm1, g1, be1):
    raise NotImplementedError("write your pallas kernel here")


# speedup vs baseline: 6.6048x; 6.6048x over previous
import dataclasses
import functools

import jax
import jax.numpy as jnp
from jax import lax
from jax.experimental import pallas as pl
from jax.experimental.pallas import tpu as pltpu
from jax.experimental.pallas import tpu_sc as plsc

N = 10000
E = 320000
D = 128
T = 9
NC = 2
NS = 16
NW = NC * NS
EPT = E // NW
CH = 80
NCHUNK = EPT // CH
NBUF = 5
RPT = 624
HR = (N * T + 127) // 128 + (1 if (N * T) % 128 else 0)
HR = 704
BLK = 1000
GRID = N // BLK

_mesh = plsc.VectorSubcoreMesh(core_axis_name="c", subcore_axis_name="s")

_sc_params = pltpu.CompilerParams()
if "needs_layout_passes" in pltpu.CompilerParams.__dataclass_fields__:
    _sc_params = dataclasses.replace(_sc_params, needs_layout_passes=False)


@functools.partial(
    pl.kernel,
    out_type=jax.ShapeDtypeStruct((NW, HR, 128), jnp.float32),
    mesh=_mesh,
    compiler_params=_sc_params,
    scratch_types=[
        pltpu.VMEM((NCHUNK, CH), jnp.int32),
        pltpu.VMEM((NCHUNK, CH), jnp.int32),
        pltpu.VMEM((HR, 128), jnp.float32),
    ],
)
def _hist_sc(dstr, typr, zro, hout, dst_v, typ_v, hist):
    c = lax.axis_index("c")
    s = lax.axis_index("s")
    w = c * NS + s
    pltpu.sync_copy(dstr.at[w], dst_v)
    pltpu.sync_copy(typr.at[w], typ_v)
    pltpu.sync_copy(zro.at[pl.ds(0, HR)], hist)
    ones = jnp.ones((16,), jnp.float32)

    @pl.loop(0, NCHUNK)
    def _(r):
        for j in range(CH // 16):
            d = dst_v[r, pl.ds(j * 16, 16)]
            t = typ_v[r, pl.ds(j * 16, 16)]
            key = d * T + t
            row = lax.shift_right_logical(key, 7)
            col = lax.bitwise_and(key, 127)
            plsc.addupdate_scatter(hist, [row, col], ones)

    pltpu.sync_copy(hist, hout.at[w])


NHALF = N // 2
AROWS = NHALF + 8
RPT2 = 312
WPT = 312


@functools.partial(
    pl.kernel,
    out_type=jax.ShapeDtypeStruct((NC, N, D), jnp.float32),
    mesh=_mesh,
    compiler_params=_sc_params,
    scratch_types=[
        pltpu.VMEM((NCHUNK, CH), jnp.int32),
        pltpu.VMEM((NCHUNK, CH), jnp.int32),
        pltpu.VMEM((2, CH), jnp.int32),
    ]
    + [pltpu.VMEM((CH, D), jnp.float32)] * NBUF
    + [pltpu.SemaphoreType.DMA] * NBUF
    + [pltpu.VMEM_SHARED((AROWS, D), jnp.float32)],
)
def _scatter_sc(x_hbm, srcr, dstr, zro, out_hbm,
                src_v, dst_v, idx_stage, b0, b1, b2, b3, b4,
                m0, m1, m2, m3, m4, acc):
    bufs = (b0, b1, b2, b3, b4)
    sems = (m0, m1, m2, m3, m4)
    c = lax.axis_index("c")
    s = lax.axis_index("s")
    w = c * NS + s
    pltpu.sync_copy(srcr.at[w], src_v)
    pltpu.sync_copy(dstr.at[w], dst_v)

    for p in (0, 1):
        pltpu.sync_copy(zro.at[pl.ds(s * RPT2, RPT2)],
                        acc.at[pl.ds(s * RPT2, RPT2)])

        @pl.when(s == NS - 1)
        def _():
            pltpu.sync_copy(zro.at[pl.ds(NS * RPT2, AROWS - NS * RPT2)],
                            acc.at[pl.ds(NS * RPT2, AROWS - NS * RPT2)])

        for b in range(NBUF):
            pltpu.async_copy(x_hbm.at[src_v.at[b]], bufs[b], sems[b])

        plsc.subcore_barrier()

        @pl.loop(0, NCHUNK, step=NBUF)
        def _(g):
            for b in range(NBUF):
                ch = g + b
                for j in range(CH // 16):
                    d = dst_v[ch, pl.ds(j * 16, 16)]
                    if p == 0:
                        adj = jnp.where(d < NHALF, d, NHALF)
                    else:
                        adj = jnp.where(d >= NHALF, d - NHALF, NHALF)
                    idx_stage[0, pl.ds(j * 16, 16)] = adj
                pltpu.make_async_copy(x_hbm.at[src_v.at[ch]],
                                      bufs[b], sems[b]).wait()
                pltpu.sync_copy(bufs[b], acc.at[idx_stage.at[0]], add=True)

                @pl.when(ch + NBUF < NCHUNK)
                def _():
                    pltpu.async_copy(x_hbm.at[src_v.at[ch + NBUF]],
                                     bufs[b], sems[b])

        plsc.subcore_barrier()
        pltpu.sync_copy(acc.at[pl.ds(s * WPT, WPT)],
                        out_hbm.at[c, pl.ds(p * NHALF + s * WPT, WPT)])

        @pl.when(s == NS - 1)
        def _():
            pltpu.sync_copy(
                acc.at[pl.ds(NS * WPT, NHALF - NS * WPT)],
                out_hbm.at[c, pl.ds(p * NHALF + NS * WPT, NHALF - NS * WPT)])


def _reduce_body(h_ref, o_ref):
    o_ref[...] = jnp.sum(h_ref[...], axis=0)


_reduce_tc = pl.pallas_call(
    _reduce_body,
    out_shape=jax.ShapeDtypeStruct((HR, 128), jnp.float32),
    grid=(8,),
    in_specs=[pl.BlockSpec((NW, HR // 8, 128), lambda i: (0, i, 0))],
    out_specs=pl.BlockSpec((HR // 8, 128), lambda i: (i, 0)),
)


def _layer_body(with_mean, x_ref, p_ref, c_ref, emb_ref, ws_ref, bs_ref,
                wm_ref, bm_ref, g_ref, be_ref, *out_refs):
    x = x_ref[...]
    agg = p_ref[0] + p_ref[1]
    agg = agg + lax.dot_general(
        c_ref[...], emb_ref[...], (((1,), (0,)), ((), ())),
        precision=lax.Precision.HIGHEST, preferred_element_type=jnp.float32)
    out = lax.dot_general(
        x, ws_ref[...], (((1,), (1,)), ((), ())),
        precision=lax.Precision.HIGHEST, preferred_element_type=jnp.float32)
    out = out + lax.dot_general(
        agg, wm_ref[...], (((1,), (1,)), ((), ())),
        precision=lax.Precision.HIGHEST, preferred_element_type=jnp.float32)
    out = out + bs_ref[...] + bm_ref[...]
    out = jnp.maximum(out, 0.0)
    mu = jnp.mean(out, axis=-1, keepdims=True)
    cen = out - mu
    var = jnp.mean(cen * cen, axis=-1, keepdims=True)
    out = cen * lax.rsqrt(var + 1e-5) * g_ref[...] + be_ref[...]
    if with_mean:
        mean_ref = out_refs[0]
        i = pl.program_id(0)

        @pl.when(i == 0)
        def _():
            mean_ref[...] = jnp.zeros_like(mean_ref)

        mean_ref[...] += jnp.sum(out, axis=0, keepdims=True) * (1.0 / N)
    else:
        out_refs[0][...] = out


_layer_in_specs = [
    pl.BlockSpec((BLK, D), lambda i: (i, 0)),
    pl.BlockSpec((NC, BLK, D), lambda i: (0, i, 0)),
    pl.BlockSpec((BLK, T), lambda i: (i, 0)),
    pl.BlockSpec((T, D), lambda i: (0, 0)),
    pl.BlockSpec((D, D), lambda i: (0, 0)),
    pl.BlockSpec((1, D), lambda i: (0, 0)),
    pl.BlockSpec((D, D), lambda i: (0, 0)),
    pl.BlockSpec((1, D), lambda i: (0, 0)),
    pl.BlockSpec((1, D), lambda i: (0, 0)),
    pl.BlockSpec((1, D), lambda i: (0, 0)),
]

_layer_tc = pl.pallas_call(
    functools.partial(_layer_body, False),
    out_shape=jax.ShapeDtypeStruct((N, D), jnp.float32),
    grid=(GRID,),
    in_specs=_layer_in_specs,
    out_specs=pl.BlockSpec((BLK, D), lambda i: (i, 0)),
)

_layer_mean_tc = pl.pallas_call(
    functools.partial(_layer_body, True),
    out_shape=jax.ShapeDtypeStruct((1, D), jnp.float32),
    grid=(GRID,),
    in_specs=_layer_in_specs,
    out_specs=pl.BlockSpec((1, D), lambda i: (0, 0)),
)


def kernel(node_embeddings, edge_index, edge_types,
           emb0, Ws0, bs0, Wm0, bm0, g0, be0,
           emb1, Ws1, bs1, Wm1, bm1, g1, be1):
    src = edge_index[0]
    dst = edge_index[1]
    dstr = dst.reshape(NW, NCHUNK, CH)
    typr = edge_types.reshape(NW, NCHUNK, CH)
    zro = jnp.zeros((N, D), jnp.float32)
    src2 = src.reshape(NW, NCHUNK, CH)

    hpart = _hist_sc(dstr, typr, zro)
    c2d = _reduce_tc(hpart)
    cn9 = c2d.reshape(-1)[: N * T].reshape(N, T)

    r1 = lambda v: v.reshape(1, D)

    x = node_embeddings
    p = _scatter_sc(x, src2, dstr, zro)
    x = _layer_tc(x, p, cn9, emb0, Ws0, r1(bs0), Wm0, r1(bm0), r1(g0), r1(be0))
    p = _scatter_sc(x, src2, dstr, zro)
    out = _layer_mean_tc(x, p, cn9, emb1, Ws1, r1(bs1), Wm1, r1(bm1),
                         r1(g1), r1(be1))
    return out.reshape(D)


# speedup vs baseline: 8.4761x; 1.2833x over previous
import dataclasses
import functools

import jax
import jax.numpy as jnp
from jax import lax
from jax.experimental import pallas as pl
from jax.experimental.pallas import tpu as pltpu
from jax.experimental.pallas import tpu_sc as plsc

N = 10000
E = 320000
D = 128
T = 9
NC = 2
NS = 16
NW = NC * NS
EPT = E // NW
CH = 80
NCHUNK = EPT // CH
NBUF = 5
RPT = 624
HR = (N * T + 127) // 128 + (1 if (N * T) % 128 else 0)
HR = 704
BLK = 1000
GRID = N // BLK

_mesh = plsc.VectorSubcoreMesh(core_axis_name="c", subcore_axis_name="s")

_sc_params = pltpu.CompilerParams()
if "needs_layout_passes" in pltpu.CompilerParams.__dataclass_fields__:
    _sc_params = dataclasses.replace(_sc_params, needs_layout_passes=False)


@functools.partial(
    pl.kernel,
    out_type=jax.ShapeDtypeStruct((NW, HR, 128), jnp.float32),
    mesh=_mesh,
    compiler_params=_sc_params,
    scratch_types=[
        pltpu.VMEM((NCHUNK, CH), jnp.int32),
        pltpu.VMEM((NCHUNK, CH), jnp.int32),
        pltpu.VMEM((HR, 128), jnp.float32),
    ],
)
def _hist_sc(dstr, typr, zro, hout, dst_v, typ_v, hist):
    c = lax.axis_index("c")
    s = lax.axis_index("s")
    w = c * NS + s
    pltpu.sync_copy(dstr.at[w], dst_v)
    pltpu.sync_copy(typr.at[w], typ_v)
    pltpu.sync_copy(zro.at[pl.ds(0, HR)], hist)
    ones = jnp.ones((16,), jnp.float32)

    @pl.loop(0, NCHUNK)
    def _(r):
        for j in range(CH // 16):
            d = dst_v[r, pl.ds(j * 16, 16)]
            t = typ_v[r, pl.ds(j * 16, 16)]
            key = d * T + t
            row = lax.shift_right_logical(key, 7)
            col = lax.bitwise_and(key, 127)
            plsc.addupdate_scatter(hist, [row, col], ones)

    pltpu.sync_copy(hist, hout.at[w])


NHALF = N // 2
AROWS = NHALF + 8
RPT2 = 312
WPT = 312


@functools.partial(
    pl.kernel,
    out_type=jax.ShapeDtypeStruct((NC, N, D), jnp.float32),
    mesh=_mesh,
    compiler_params=_sc_params,
    scratch_types=[
        pltpu.VMEM((NCHUNK, CH), jnp.int32),
        pltpu.VMEM((NCHUNK, CH), jnp.int32),
        pltpu.VMEM((NBUF, CH), jnp.int32),
    ]
    + [pltpu.VMEM((CH, D), jnp.float32)] * NBUF
    + [pltpu.SemaphoreType.DMA] * NBUF
    + [pltpu.SemaphoreType.DMA] * NBUF
    + [pltpu.VMEM_SHARED((AROWS, D), jnp.float32)],
)
def _scatter_sc(x_hbm, srcr, dstr, zro, out_hbm,
                src_v, dst_v, idx_stage, b0, b1, b2, b3, b4,
                m0, m1, m2, m3, m4, n0, n1, n2, n3, n4, acc):
    bufs = (b0, b1, b2, b3, b4)
    gsem = (m0, m1, m2, m3, m4)
    ssem = (n0, n1, n2, n3, n4)
    c = lax.axis_index("c")
    s = lax.axis_index("s")
    w = c * NS + s
    pltpu.sync_copy(srcr.at[w], src_v)
    pltpu.sync_copy(dstr.at[w], dst_v)

    GPRE = 3

    for p in (0, 1):
        pltpu.sync_copy(zro.at[pl.ds(s * RPT2, RPT2)],
                        acc.at[pl.ds(s * RPT2, RPT2)])

        @pl.when(s == NS - 1)
        def _():
            pltpu.sync_copy(zro.at[pl.ds(NS * RPT2, AROWS - NS * RPT2)],
                            acc.at[pl.ds(NS * RPT2, AROWS - NS * RPT2)])

        for b in range(GPRE):
            pltpu.async_copy(x_hbm.at[src_v.at[b]], bufs[b], gsem[b])

        plsc.subcore_barrier()

        @pl.loop(0, NCHUNK, step=NBUF)
        def _(g):
            for b in range(NBUF):
                ch = g + b
                for j in range(CH // 16):
                    d = dst_v[ch, pl.ds(j * 16, 16)]
                    grb = NHALF + lax.bitwise_and(d, 7)
                    if p == 0:
                        adj = jnp.where(d < NHALF, d, grb)
                    else:
                        adj = jnp.where(d >= NHALF, d - NHALF, grb)
                    idx_stage[b, pl.ds(j * 16, 16)] = adj
                pltpu.make_async_copy(x_hbm.at[src_v.at[ch]],
                                      bufs[b], gsem[b]).wait()
                pltpu.async_copy(bufs[b], acc.at[idx_stage.at[b]],
                                 ssem[b], add=True)
                b2 = (b + NBUF - 2) % NBUF

                @pl.when(ch >= 2)
                def _():
                    pltpu.make_async_copy(bufs[b2], acc.at[idx_stage.at[b2]],
                                          ssem[b2]).wait()

                b3 = (b + GPRE) % NBUF

                @pl.when(ch + GPRE < NCHUNK)
                def _():
                    pltpu.async_copy(x_hbm.at[src_v.at[ch + GPRE]],
                                     bufs[b3], gsem[b3])

        for b2 in ((NCHUNK - 2) % NBUF, (NCHUNK - 1) % NBUF):
            pltpu.make_async_copy(bufs[b2], acc.at[idx_stage.at[b2]],
                                  ssem[b2]).wait()

        plsc.subcore_barrier()
        pltpu.sync_copy(acc.at[pl.ds(s * WPT, WPT)],
                        out_hbm.at[c, pl.ds(p * NHALF + s * WPT, WPT)])

        @pl.when(s == NS - 1)
        def _():
            pltpu.sync_copy(
                acc.at[pl.ds(NS * WPT, NHALF - NS * WPT)],
                out_hbm.at[c, pl.ds(p * NHALF + NS * WPT, NHALF - NS * WPT)])


def _reduce_body(h_ref, o_ref):
    o_ref[...] = jnp.sum(h_ref[...], axis=0)


_reduce_tc = pl.pallas_call(
    _reduce_body,
    out_shape=jax.ShapeDtypeStruct((HR, 128), jnp.float32),
    grid=(8,),
    in_specs=[pl.BlockSpec((NW, HR // 8, 128), lambda i: (0, i, 0))],
    out_specs=pl.BlockSpec((HR // 8, 128), lambda i: (i, 0)),
)


def _layer_body(with_mean, x_ref, p_ref, c_ref, emb_ref, ws_ref, bs_ref,
                wm_ref, bm_ref, g_ref, be_ref, *out_refs):
    x = x_ref[...]
    agg = p_ref[0] + p_ref[1]
    agg = agg + lax.dot_general(
        c_ref[...], emb_ref[...], (((1,), (0,)), ((), ())),
        precision=lax.Precision.HIGHEST, preferred_element_type=jnp.float32)
    out = lax.dot_general(
        x, ws_ref[...], (((1,), (1,)), ((), ())),
        precision=lax.Precision.HIGHEST, preferred_element_type=jnp.float32)
    out = out + lax.dot_general(
        agg, wm_ref[...], (((1,), (1,)), ((), ())),
        precision=lax.Precision.HIGHEST, preferred_element_type=jnp.float32)
    out = out + bs_ref[...] + bm_ref[...]
    out = jnp.maximum(out, 0.0)
    mu = jnp.mean(out, axis=-1, keepdims=True)
    cen = out - mu
    var = jnp.mean(cen * cen, axis=-1, keepdims=True)
    out = cen * lax.rsqrt(var + 1e-5) * g_ref[...] + be_ref[...]
    if with_mean:
        mean_ref = out_refs[0]
        i = pl.program_id(0)

        @pl.when(i == 0)
        def _():
            mean_ref[...] = jnp.zeros_like(mean_ref)

        mean_ref[...] += jnp.sum(out, axis=0, keepdims=True) * (1.0 / N)
    else:
        out_refs[0][...] = out


_layer_in_specs = [
    pl.BlockSpec((BLK, D), lambda i: (i, 0)),
    pl.BlockSpec((NC, BLK, D), lambda i: (0, i, 0)),
    pl.BlockSpec((BLK, T), lambda i: (i, 0)),
    pl.BlockSpec((T, D), lambda i: (0, 0)),
    pl.BlockSpec((D, D), lambda i: (0, 0)),
    pl.BlockSpec((1, D), lambda i: (0, 0)),
    pl.BlockSpec((D, D), lambda i: (0, 0)),
    pl.BlockSpec((1, D), lambda i: (0, 0)),
    pl.BlockSpec((1, D), lambda i: (0, 0)),
    pl.BlockSpec((1, D), lambda i: (0, 0)),
]

_layer_tc = pl.pallas_call(
    functools.partial(_layer_body, False),
    out_shape=jax.ShapeDtypeStruct((N, D), jnp.float32),
    grid=(GRID,),
    in_specs=_layer_in_specs,
    out_specs=pl.BlockSpec((BLK, D), lambda i: (i, 0)),
)

_layer_mean_tc = pl.pallas_call(
    functools.partial(_layer_body, True),
    out_shape=jax.ShapeDtypeStruct((1, D), jnp.float32),
    grid=(GRID,),
    in_specs=_layer_in_specs,
    out_specs=pl.BlockSpec((1, D), lambda i: (0, 0)),
)


def kernel(node_embeddings, edge_index, edge_types,
           emb0, Ws0, bs0, Wm0, bm0, g0, be0,
           emb1, Ws1, bs1, Wm1, bm1, g1, be1):
    src = edge_index[0]
    dst = edge_index[1]
    dstr = dst.reshape(NW, NCHUNK, CH)
    typr = edge_types.reshape(NW, NCHUNK, CH)
    zro = jnp.zeros((N, D), jnp.float32)
    src2 = src.reshape(NW, NCHUNK, CH)

    hpart = _hist_sc(dstr, typr, zro)
    c2d = _reduce_tc(hpart)
    cn9 = c2d.reshape(-1)[: N * T].reshape(N, T)

    r1 = lambda v: v.reshape(1, D)

    x = node_embeddings
    p = _scatter_sc(x, src2, dstr, zro)
    x = _layer_tc(x, p, cn9, emb0, Ws0, r1(bs0), Wm0, r1(bm0), r1(g0), r1(be0))
    p = _scatter_sc(x, src2, dstr, zro)
    out = _layer_mean_tc(x, p, cn9, emb1, Ws1, r1(bs1), Wm1, r1(bm1),
                         r1(g1), r1(be1))
    return out.reshape(D)


# speedup vs baseline: 8.7957x; 1.0377x over previous
import dataclasses
import functools

import jax
import jax.numpy as jnp
from jax import lax
from jax.experimental import pallas as pl
from jax.experimental.pallas import tpu as pltpu
from jax.experimental.pallas import tpu_sc as plsc

N = 10000
E = 320000
D = 128
T = 9
NC = 2
NS = 16
NW = NC * NS
EPT = E // NW
CH = 80
NCHUNK = EPT // CH
NBUF = 5
RPT = 624
HR = (N * T + 127) // 128 + (1 if (N * T) % 128 else 0)
HR = 704
BLK = 1000
GRID = N // BLK

_mesh = plsc.VectorSubcoreMesh(core_axis_name="c", subcore_axis_name="s")

_sc_params = pltpu.CompilerParams()
if "needs_layout_passes" in pltpu.CompilerParams.__dataclass_fields__:
    _sc_params = dataclasses.replace(_sc_params, needs_layout_passes=False)


@functools.partial(
    pl.kernel,
    out_type=jax.ShapeDtypeStruct((NW, HR, 128), jnp.float32),
    mesh=_mesh,
    compiler_params=_sc_params,
    scratch_types=[
        pltpu.VMEM((NCHUNK, CH), jnp.int32),
        pltpu.VMEM((NCHUNK, CH), jnp.int32),
        pltpu.VMEM((HR, 128), jnp.float32),
    ],
)
def _hist_sc(dstr, typr, zro, hout, dst_v, typ_v, hist):
    c = lax.axis_index("c")
    s = lax.axis_index("s")
    w = c * NS + s
    pltpu.sync_copy(dstr.at[w], dst_v)
    pltpu.sync_copy(typr.at[w], typ_v)
    pltpu.sync_copy(zro.at[pl.ds(0, HR)], hist)
    ones = jnp.ones((16,), jnp.float32)

    @pl.loop(0, NCHUNK)
    def _(r):
        for j in range(CH // 16):
            d = dst_v[r, pl.ds(j * 16, 16)]
            t = typ_v[r, pl.ds(j * 16, 16)]
            key = d * T + t
            row = lax.shift_right_logical(key, 7)
            col = lax.bitwise_and(key, 127)
            plsc.addupdate_scatter(hist, [row, col], ones)

    pltpu.sync_copy(hist, hout.at[w])


NHALF = N // 2
AROWS = NHALF + 8
RPT2 = 312
WPT = 312


@functools.partial(
    pl.kernel,
    out_type=jax.ShapeDtypeStruct((NC, N, D), jnp.float32),
    mesh=_mesh,
    compiler_params=_sc_params,
    scratch_types=[
        pltpu.VMEM((NCHUNK, CH), jnp.int32),
        pltpu.VMEM((NCHUNK, CH), jnp.int32),
        pltpu.VMEM((NBUF, CH), jnp.int32),
    ]
    + [pltpu.VMEM((CH, D), jnp.float32)] * NBUF
    + [pltpu.SemaphoreType.DMA] * NBUF
    + [pltpu.SemaphoreType.DMA] * NBUF
    + [pltpu.VMEM_SHARED((AROWS, D), jnp.float32)],
)
def _scatter_sc(x_hbm, srcr, dstr, zro, out_hbm,
                src_v, dst_v, idx_stage, b0, b1, b2, b3, b4,
                m0, m1, m2, m3, m4, n0, n1, n2, n3, n4, acc):
    bufs = (b0, b1, b2, b3, b4)
    gsem = (m0, m1, m2, m3, m4)
    ssem = (n0, n1, n2, n3, n4)
    c = lax.axis_index("c")
    s = lax.axis_index("s")
    w = c * NS + s
    pltpu.sync_copy(srcr.at[w], src_v)
    pltpu.sync_copy(dstr.at[w], dst_v)

    GPRE = 4

    for p in (0, 1):
        pltpu.sync_copy(zro.at[pl.ds(s * RPT2, RPT2)],
                        acc.at[pl.ds(s * RPT2, RPT2)])

        @pl.when(s == NS - 1)
        def _():
            pltpu.sync_copy(zro.at[pl.ds(NS * RPT2, AROWS - NS * RPT2)],
                            acc.at[pl.ds(NS * RPT2, AROWS - NS * RPT2)])

        for b in range(GPRE):
            pltpu.async_copy(x_hbm.at[src_v.at[b]], bufs[b], gsem[b])

        plsc.subcore_barrier()

        @pl.loop(0, NCHUNK, step=NBUF)
        def _(g):
            for b in range(NBUF):
                ch = g + b
                for j in range(CH // 16):
                    d = dst_v[ch, pl.ds(j * 16, 16)]
                    grb = NHALF + lax.bitwise_and(d, 7)
                    if p == 0:
                        adj = jnp.where(d < NHALF, d, grb)
                    else:
                        adj = jnp.where(d >= NHALF, d - NHALF, grb)
                    idx_stage[b, pl.ds(j * 16, 16)] = adj
                pltpu.make_async_copy(x_hbm.at[src_v.at[ch]],
                                      bufs[b], gsem[b]).wait()
                pltpu.async_copy(bufs[b], acc.at[idx_stage.at[b]],
                                 ssem[b], add=True)
                b2 = (b + NBUF - (NBUF - GPRE)) % NBUF

                @pl.when(ch >= NBUF - GPRE)
                def _():
                    pltpu.make_async_copy(bufs[b2], acc.at[idx_stage.at[b2]],
                                          ssem[b2]).wait()

                b3 = (b + GPRE) % NBUF

                @pl.when(ch + GPRE < NCHUNK)
                def _():
                    pltpu.async_copy(x_hbm.at[src_v.at[ch + GPRE]],
                                     bufs[b3], gsem[b3])

        for k in range(NBUF - GPRE):
            b2 = (NCHUNK - 1 - k) % NBUF
            pltpu.make_async_copy(bufs[b2], acc.at[idx_stage.at[b2]],
                                  ssem[b2]).wait()

        plsc.subcore_barrier()
        pltpu.sync_copy(acc.at[pl.ds(s * WPT, WPT)],
                        out_hbm.at[c, pl.ds(p * NHALF + s * WPT, WPT)])

        @pl.when(s == NS - 1)
        def _():
            pltpu.sync_copy(
                acc.at[pl.ds(NS * WPT, NHALF - NS * WPT)],
                out_hbm.at[c, pl.ds(p * NHALF + NS * WPT, NHALF - NS * WPT)])


def _reduce_body(h_ref, o_ref):
    o_ref[...] = jnp.sum(h_ref[...], axis=0)


_reduce_tc = pl.pallas_call(
    _reduce_body,
    out_shape=jax.ShapeDtypeStruct((HR, 128), jnp.float32),
    grid=(8,),
    in_specs=[pl.BlockSpec((NW, HR // 8, 128), lambda i: (0, i, 0))],
    out_specs=pl.BlockSpec((HR // 8, 128), lambda i: (i, 0)),
)


def _layer_body(with_mean, x_ref, p_ref, c_ref, emb_ref, ws_ref, bs_ref,
                wm_ref, bm_ref, g_ref, be_ref, *out_refs):
    x = x_ref[...]
    agg = p_ref[0] + p_ref[1]
    agg = agg + lax.dot_general(
        c_ref[...], emb_ref[...], (((1,), (0,)), ((), ())),
        precision=lax.Precision.HIGHEST, preferred_element_type=jnp.float32)
    out = lax.dot_general(
        x, ws_ref[...], (((1,), (1,)), ((), ())),
        precision=lax.Precision.HIGHEST, preferred_element_type=jnp.float32)
    out = out + lax.dot_general(
        agg, wm_ref[...], (((1,), (1,)), ((), ())),
        precision=lax.Precision.HIGHEST, preferred_element_type=jnp.float32)
    out = out + bs_ref[...] + bm_ref[...]
    out = jnp.maximum(out, 0.0)
    mu = jnp.mean(out, axis=-1, keepdims=True)
    cen = out - mu
    var = jnp.mean(cen * cen, axis=-1, keepdims=True)
    out = cen * lax.rsqrt(var + 1e-5) * g_ref[...] + be_ref[...]
    if with_mean:
        mean_ref = out_refs[0]
        i = pl.program_id(0)

        @pl.when(i == 0)
        def _():
            mean_ref[...] = jnp.zeros_like(mean_ref)

        mean_ref[...] += jnp.sum(out, axis=0, keepdims=True) * (1.0 / N)
    else:
        out_refs[0][...] = out


_layer_in_specs = [
    pl.BlockSpec((BLK, D), lambda i: (i, 0)),
    pl.BlockSpec((NC, BLK, D), lambda i: (0, i, 0)),
    pl.BlockSpec((BLK, T), lambda i: (i, 0)),
    pl.BlockSpec((T, D), lambda i: (0, 0)),
    pl.BlockSpec((D, D), lambda i: (0, 0)),
    pl.BlockSpec((1, D), lambda i: (0, 0)),
    pl.BlockSpec((D, D), lambda i: (0, 0)),
    pl.BlockSpec((1, D), lambda i: (0, 0)),
    pl.BlockSpec((1, D), lambda i: (0, 0)),
    pl.BlockSpec((1, D), lambda i: (0, 0)),
]

_layer_tc = pl.pallas_call(
    functools.partial(_layer_body, False),
    out_shape=jax.ShapeDtypeStruct((N, D), jnp.float32),
    grid=(GRID,),
    in_specs=_layer_in_specs,
    out_specs=pl.BlockSpec((BLK, D), lambda i: (i, 0)),
)

_layer_mean_tc = pl.pallas_call(
    functools.partial(_layer_body, True),
    out_shape=jax.ShapeDtypeStruct((1, D), jnp.float32),
    grid=(GRID,),
    in_specs=_layer_in_specs,
    out_specs=pl.BlockSpec((1, D), lambda i: (0, 0)),
)


def kernel(node_embeddings, edge_index, edge_types,
           emb0, Ws0, bs0, Wm0, bm0, g0, be0,
           emb1, Ws1, bs1, Wm1, bm1, g1, be1):
    src = edge_index[0]
    dst = edge_index[1]
    dstr = dst.reshape(NW, NCHUNK, CH)
    typr = edge_types.reshape(NW, NCHUNK, CH)
    zro = jnp.zeros((N, D), jnp.float32)
    src2 = src.reshape(NW, NCHUNK, CH)

    hpart = _hist_sc(dstr, typr, zro)
    c2d = _reduce_tc(hpart)
    cn9 = c2d.reshape(-1)[: N * T].reshape(N, T)

    r1 = lambda v: v.reshape(1, D)

    x = node_embeddings
    p = _scatter_sc(x, src2, dstr, zro)
    x = _layer_tc(x, p, cn9, emb0, Ws0, r1(bs0), Wm0, r1(bm0), r1(g0), r1(be0))
    p = _scatter_sc(x, src2, dstr, zro)
    out = _layer_mean_tc(x, p, cn9, emb1, Ws1, r1(bs1), Wm1, r1(bm1),
                         r1(g1), r1(be1))
    return out.reshape(D)
